# Initial kernel scaffold; baseline (speedup 1.0000x reference)
import jax
import jax.numpy as jnp
from jax import lax
from jax.experimental import pallas as pl
from jax.experimental.pallas import tpu as pltpu
from jax.experimental.pallas import tpu_sc as plsc

N = 50000
E = 800000
G = 512
NP = 50048
NCHN = NP // 128
CH = 128
NCHE = E // CH
CPW = NCHE // 32
EXTRA = NCHE - 32 * CPW
GP = 520

_f32 = jnp.float32
_i32 = jnp.int32


def _mesh():
    return plsc.VectorSubcoreMesh(core_axis_name="c", subcore_axis_name="s")


def _wids():
    c = lax.axis_index("c")
    s = lax.axis_index("s")
    return c, s, c * 16 + s


def _zero_acc(zbuf, acc_sh, s):
    nzk = 24 + (s < 7).astype(_i32)

    def zbody(i, _):
        k = s + 16 * i
        pltpu.sync_copy(zbuf, acc_sh.at[pl.ds(k * 128, 128)])
        return 0

    lax.fori_loop(0, nzk, zbody, 0)


def _copy_out_acc(acc_sh, stage, out_hbm, c, s):
    nzk = 24 + (s < 7).astype(_i32)

    def obody(i, _):
        k = s + 16 * i
        pltpu.sync_copy(acc_sh.at[pl.ds(k * 128, 128)], stage)
        pltpu.sync_copy(stage, out_hbm.at[c, pl.ds(k * 128, 128)])
        return 0

    lax.fori_loop(0, nzk, obody, 0)


def _load_edge_chunks(e2d_hbm, buf, wid):
    pltpu.sync_copy(e2d_hbm.at[pl.ds(wid * CPW, CPW)], buf.at[pl.ds(0, CPW)])

    @pl.when(wid < EXTRA)
    def _():
        pltpu.sync_copy(e2d_hbm.at[32 * CPW + wid], buf.at[CPW])


def _deg_body(dst_hbm, ones_hbm, zeros_hbm, out_hbm,
              acc_sh, idx_all, ones_v, zbuf):
    c, s, wid = _wids()
    pltpu.sync_copy(ones_hbm, ones_v)
    pltpu.sync_copy(zeros_hbm, zbuf)
    _zero_acc(zbuf, acc_sh, s)
    plsc.subcore_barrier()
    _load_edge_chunks(dst_hbm, idx_all, wid)
    nch = CPW + (wid < EXTRA).astype(_i32)

    def ebody(j, _):
        pltpu.sync_copy(ones_v, acc_sh.at[idx_all.at[j]], add=True)
        return 0

    lax.fori_loop(0, nch, ebody, 0)
    plsc.subcore_barrier()
    _copy_out_acc(acc_sh, zbuf, out_hbm, c, s)


def _deg_call(dst2d, ones_t, zeros_t):
    return pl.kernel(
        _deg_body,
        out_type=jax.ShapeDtypeStruct((2, NP, 4), _f32),
        mesh=_mesh(),
        scratch_types=[
            pltpu.VMEM_SHARED((NP, 4), _f32),
            pltpu.VMEM((CPW + 1, CH), _i32),
            pltpu.VMEM((CH, 4), _f32),
            pltpu.VMEM((CH, 4), _f32),
        ],
    )(dst2d, ones_t, zeros_t)


def _step_body(u_hbm, src_hbm, dst_hbm, zeros_hbm, out_hbm,
               acc_sh, src_all, dst_all, rows, zbuf, sem0, sem1):
    c, s, wid = _wids()
    pltpu.sync_copy(zeros_hbm, zbuf)
    _zero_acc(zbuf, acc_sh, s)
    plsc.subcore_barrier()
    _load_edge_chunks(src_hbm, src_all, wid)
    _load_edge_chunks(dst_hbm, dst_all, wid)
    nch = CPW + (wid < EXTRA).astype(_i32)
    r0 = rows.at[0]
    r1 = rows.at[1]

    def fire(ch, rbuf, sem):
        pltpu.async_copy(u_hbm.at[src_all.at[ch]], rbuf, sem)

    def waitg(rbuf, sem):
        pltpu.make_async_copy(u_hbm.at[src_all.at[0]], rbuf, sem).wait()

    def scat(ch, rbuf):
        pltpu.sync_copy(rbuf, acc_sh.at[dst_all.at[ch]], add=True)

    fire(0, r0, sem0)
    npair = nch // 2

    def pbody(i, _):
        ch0 = 2 * i
        waitg(r0, sem0)
        fire(ch0 + 1, r1, sem1)
        scat(ch0, r0)
        waitg(r1, sem1)

        @pl.when(ch0 + 2 < nch)
        def _():
            fire(ch0 + 2, r0, sem0)

        scat(ch0 + 1, r1)
        return 0

    lax.fori_loop(0, npair, pbody, 0)

    @pl.when(wid >= EXTRA)
    def _():
        waitg(r0, sem0)
        scat(CPW - 1, r0)

    plsc.subcore_barrier()
    _copy_out_acc(acc_sh, zbuf, out_hbm, c, s)


def _step_call(u, src2d, dst2d, zeros_t):
    return pl.kernel(
        _step_body,
        out_type=jax.ShapeDtypeStruct((2, NP, 4), _f32),
        mesh=_mesh(),
        scratch_types=[
            pltpu.VMEM_SHARED((NP, 4), _f32),
            pltpu.VMEM((CPW + 1, CH), _i32),
            pltpu.VMEM((CPW + 1, CH), _i32),
            pltpu.VMEM((2, CH, 4), _f32),
            pltpu.VMEM((CH, 4), _f32),
            pltpu.SemaphoreType.DMA,
            pltpu.SemaphoreType.DMA,
        ],
    )(u, src2d, dst2d, zeros_t)


def _merge_body(acc_hbm, u_hbm, s4_hbm, d1_hbm, un_hbm, zc_hbm,
                abuf, bbuf, ubuf, sbuf, sumbuf, obuf, dbuf, zbuf):
    _, _, wid = _wids()
    nmk = 12 + (wid < 7).astype(_i32)
    iota = lax.iota(_i32, 16)

    def body(i, _):
        k = wid + 32 * i
        foff = k * 512
        noff = k * 128
        pltpu.sync_copy(acc_hbm.at[0, pl.ds(foff, 512)], abuf)
        pltpu.sync_copy(acc_hbm.at[1, pl.ds(foff, 512)], bbuf)
        pltpu.sync_copy(u_hbm.at[pl.ds(foff, 512)], ubuf)
        pltpu.sync_copy(s4_hbm.at[pl.ds(foff, 512)], sbuf)
        pltpu.sync_copy(d1_hbm.at[pl.ds(noff, 128)], dbuf)

        def rbody(r, _):
            sl = pl.ds(r * 16, 16)
            sm = abuf[sl] + bbuf[sl] + ubuf[sl]
            sumbuf[sl] = sm
            obuf[sl] = sm * sbuf[sl]
            return 0

        lax.fori_loop(0, 32, rbody, 0)

        def zbody(r, _):
            idx = (r * 16 + iota) * 4 + 3
            sv = plsc.load_gather(sumbuf, [idx])
            zbuf[pl.ds(r * 16, 16)] = dbuf[pl.ds(r * 16, 16)] * sv
            return 0

        lax.fori_loop(0, 8, zbody, 0)
        pltpu.sync_copy(obuf, un_hbm.at[pl.ds(foff, 512)])
        pltpu.sync_copy(zbuf, zc_hbm.at[pl.ds(noff, 128)])
        return 0

    lax.fori_loop(0, nmk, body, 0)


def _merge_call(acc2f, uf, s4f, d1):
    return pl.kernel(
        _merge_body,
        out_type=(jax.ShapeDtypeStruct((NP * 4,), _f32),
                  jax.ShapeDtypeStruct((NP,), _f32)),
        mesh=_mesh(),
        scratch_types=[
            pltpu.VMEM((512,), _f32),
            pltpu.VMEM((512,), _f32),
            pltpu.VMEM((512,), _f32),
            pltpu.VMEM((512,), _f32),
            pltpu.VMEM((512,), _f32),
            pltpu.VMEM((512,), _f32),
            pltpu.VMEM((128,), _f32),
            pltpu.VMEM((128,), _f32),
        ],
    )(acc2f, uf, s4f, d1)


def _seg_body(z3_hbm, zc1_hbm, zc2_hbm, b_hbm, zeros_hbm,
              s4_out, sz_out,
              acc4_sh, accz_sh, z3buf, zrow, c1buf, c2buf, bidx, stage):
    c, s, wid = _wids()
    iota = lax.iota(_i32, 16)
    full0 = jnp.full((16,), 0, _i32)
    full1 = jnp.full((16,), 1, _i32)
    full2 = jnp.full((16,), 2, _i32)
    full3 = jnp.full((16,), 3, _i32)
    ones16 = jnp.full((16,), 1.0, _f32)
    zero16 = jnp.full((16,), 0.0, _f32)
    pltpu.sync_copy(zeros_hbm, stage)

    @pl.when(s == 0)
    def _():
        for k in range(4):
            pltpu.sync_copy(stage, acc4_sh.at[pl.ds(k * 128, 128)])
            pltpu.sync_copy(stage, accz_sh.at[pl.ds(k * 128, 128)])
        pltpu.sync_copy(stage.at[pl.ds(0, 8)], acc4_sh.at[pl.ds(512, 8)])
        pltpu.sync_copy(stage.at[pl.ds(0, 8)], accz_sh.at[pl.ds(512, 8)])

    def cbody(r, _):
        ridx = r * 16 + iota
        plsc.store_scatter(zrow, [ridx, full2], ones16)
        plsc.store_scatter(zrow, [ridx, full3], zero16)
        return 0

    lax.fori_loop(0, 8, cbody, 0)
    plsc.subcore_barrier()
    nmk = 12 + (wid < 7).astype(_i32)

    def body(i, _):
        k = wid + 32 * i
        noff = k * 128
        pltpu.sync_copy(b_hbm.at[pl.ds(noff, 128)], bidx)
        pltpu.sync_copy(z3_hbm.at[pl.ds(noff // 128 * 128, 128)], z3buf)
        pltpu.sync_copy(zc1_hbm.at[pl.ds(noff, 128)], c1buf)
        pltpu.sync_copy(zc2_hbm.at[pl.ds(noff, 128)], c2buf)

        def fbody(r, _):
            ridx = r * 16 + iota
            plsc.store_scatter(zrow, [ridx, full0], c1buf[pl.ds(r * 16, 16)])
            plsc.store_scatter(zrow, [ridx, full1], c2buf[pl.ds(r * 16, 16)])
            return 0

        lax.fori_loop(0, 8, fbody, 0)
        pltpu.sync_copy(z3buf, acc4_sh.at[bidx], add=True)
        pltpu.sync_copy(zrow, accz_sh.at[bidx], add=True)
        return 0

    lax.fori_loop(0, nmk, body, 0)
    plsc.subcore_barrier()

    @pl.when(s == 0)
    def _():
        for k in range(4):
            pltpu.sync_copy(acc4_sh.at[pl.ds(k * 128, 128)], stage)
            pltpu.sync_copy(stage, s4_out.at[c, pl.ds(k * 128, 128)])
            pltpu.sync_copy(accz_sh.at[pl.ds(k * 128, 128)], stage)
            pltpu.sync_copy(stage, sz_out.at[c, pl.ds(k * 128, 128)])
        pltpu.sync_copy(acc4_sh.at[pl.ds(512, 8)], stage.at[pl.ds(0, 8)])
        pltpu.sync_copy(stage.at[pl.ds(0, 8)], s4_out.at[c, pl.ds(512, 8)])
        pltpu.sync_copy(accz_sh.at[pl.ds(512, 8)], stage.at[pl.ds(0, 8)])
        pltpu.sync_copy(stage.at[pl.ds(0, 8)], sz_out.at[c, pl.ds(512, 8)])


def _seg_call(z3, zc1, zc2, bp, zeros_t):
    return pl.kernel(
        _seg_body,
        out_type=(jax.ShapeDtypeStruct((2, GP, 4), _f32),
                  jax.ShapeDtypeStruct((2, GP, 4), _f32)),
        mesh=_mesh(),
        scratch_types=[
            pltpu.VMEM_SHARED((GP, 4), _f32),
            pltpu.VMEM_SHARED((GP, 4), _f32),
            pltpu.VMEM((CH, 4), _f32),
            pltpu.VMEM((CH, 4), _f32),
            pltpu.VMEM((CH,), _f32),
            pltpu.VMEM((CH,), _f32),
            pltpu.VMEM((CH,), _i32),
            pltpu.VMEM((CH, 4), _f32),
        ],
    )(z3, zc1, zc2, bp, zeros_t)


_RB = NP // 16


def _prep_body(degs_ref, x_ref, u1_ref, d24_ref, d14_ref, d1_ref):
    d = degs_ref[0] + degs_ref[1]
    deg = d[:, 0:1] + 1.0
    dinv = lax.rsqrt(deg)
    d2 = dinv * dinv
    ones = jnp.ones_like(deg)
    u1_ref[...] = jnp.concatenate([x_ref[...], ones], axis=1) * dinv
    d24_ref[...] = jnp.broadcast_to(d2, d.shape)
    d14_ref[...] = jnp.broadcast_to(dinv, d.shape)
    d1_ref[...] = dinv


def _prep_call(degs, xp):
    return pl.pallas_call(
        _prep_body,
        grid=(NP // _RB,),
        in_specs=[
            pl.BlockSpec((2, _RB, 4), lambda i: (0, i, 0)),
            pl.BlockSpec((_RB, 3), lambda i: (i, 0)),
        ],
        out_specs=[
            pl.BlockSpec((_RB, 4), lambda i: (i, 0)),
            pl.BlockSpec((_RB, 4), lambda i: (i, 0)),
            pl.BlockSpec((_RB, 4), lambda i: (i, 0)),
            pl.BlockSpec((_RB, 1), lambda i: (i, 0)),
        ],
        out_shape=[
            jax.ShapeDtypeStruct((NP, 4), _f32),
            jax.ShapeDtypeStruct((NP, 4), _f32),
            jax.ShapeDtypeStruct((NP, 4), _f32),
            jax.ShapeDtypeStruct((NP, 1), _f32),
        ],
    )(degs, xp)


def _final_body(s4_ref, sz_ref, w1_ref, b1_ref, w2_ref, b2_ref,
                w3_ref, b3_ref, out_ref):
    s4 = s4_ref[0] + s4_ref[1]
    sz = sz_ref[0] + sz_ref[1]
    m3 = s4[:512, 0:3]
    z1s = sz[:512, 0:1]
    z2s = sz[:512, 1:2]
    cnt = jnp.maximum(sz[:512, 2:3], 1.0)
    w12 = jnp.dot(w1_ref[...], w2_ref[...], preferred_element_type=_f32)
    w123 = jnp.dot(w12, w3_ref[...], preferred_element_type=_f32)
    v1 = jnp.dot(jnp.dot(b1_ref[...], w2_ref[...],
                         preferred_element_type=_f32),
                 w3_ref[...], preferred_element_type=_f32)
    v2 = jnp.dot(b2_ref[...], w3_ref[...], preferred_element_type=_f32)
    num = jnp.dot(m3, w123, preferred_element_type=_f32)
    num = num + z2s * v1 + z1s * v2
    out_ref[...] = num / cnt + b3_ref[...]


def _final_call(s4, sz, W1, b1, W2, b2, W3, b3):
    return pl.pallas_call(
        _final_body,
        out_shape=jax.ShapeDtypeStruct((G, 192), _f32),
    )(s4, sz, W1, b1, W2, b2, W3, b3)


def kernel(node_features, edge_indices, batch_indices,
           W1, b1, W2, b2, W3, b3):
    src2d = edge_indices[0].reshape(NCHE, CH)
    dst2d = edge_indices[1].reshape(NCHE, CH)
    xp = jnp.pad(node_features, ((0, NP - N), (0, 0)))
    bp = jnp.pad(batch_indices, (0, NP - N), constant_values=G)
    zeros_t = jnp.zeros((CH, 4), _f32)
    ones_t = jnp.ones((CH, 4), _f32)

    degs = _deg_call(dst2d, ones_t, zeros_t)
    u1, d24, d14, d1c = _prep_call(degs, xp)
    d24f = d24.reshape(-1)
    d14f = d14.reshape(-1)
    d1 = d1c.reshape(-1)

    u = u1
    zcs = []
    for t in range(3):
        acc = _step_call(u, src2d, dst2d, zeros_t)
        sf = d24f if t < 2 else d14f
        unf, zc = _merge_call(acc.reshape(2, -1), u.reshape(-1), sf, d1)
        u = unf.reshape(NP, 4)
        zcs.append(zc)

    s4, sz = _seg_call(u, zcs[0], zcs[1], bp, zeros_t)
    out = _final_call(s4, sz, W1, b1.reshape(1, -1), W2,
                      b2.reshape(1, -1), W3, b3.reshape(1, -1))
    return out


# speedup vs baseline: 66.4453x; 66.4453x over previous
import jax
import jax.numpy as jnp
from jax import lax
from jax.experimental import pallas as pl
from jax.experimental.pallas import tpu as pltpu
from jax.experimental.pallas import tpu_sc as plsc

N = 50000
E = 800000
G = 512
NP = 50176
EP = 819200
EPR = EP // 8
EPW = EP // 32
K = 2048
KD = 2560
GP = 520
GW = 8

_f32 = jnp.float32
_i32 = jnp.int32

_SC_PARAMS = pltpu.CompilerParams(use_tc_tiling_on_sc=False,
                                  needs_layout_passes=False)


def _mesh():
    return plsc.VectorSubcoreMesh(core_axis_name="c", subcore_axis_name="s")


def _kw():
    return dict(mesh=_mesh(), compiler_params=_SC_PARAMS)


def _wid():
    return lax.axis_index("c") * 16 + lax.axis_index("s")


def _zero(buf, nwords):
    z = jnp.zeros((16,), _f32)

    def zb(i, _):
        buf[pl.ds(i * 16, 16)] = z
        return 0

    lax.fori_loop(0, nwords // 16, zb, 0)


def _deg_body(dst_hbm, out_hbm, acc, b0, b1, s0, s1):
    wid = _wid()
    _zero(acc, NP)
    ones16 = jnp.full((16,), 1.0, _f32)
    base = wid * EPW
    nch = EPW // KD
    bufs = (b0, b1)
    sems = (s0, s1)

    def fire(ch):
        pltpu.async_copy(dst_hbm.at[pl.ds(base + ch * KD, KD)],
                         bufs[ch % 2], sems[ch % 2])

    fire(0)
    for ch in range(nch):
        buf, sem = bufs[ch % 2], sems[ch % 2]
        pltpu.make_async_copy(dst_hbm.at[pl.ds(base + ch * KD, KD)],
                              buf, sem).wait()
        if ch + 1 < nch:
            fire(ch + 1)

        def eb(i, _):
            plsc.addupdate_scatter(acc, [buf[pl.ds(i * 16, 16)]], ones16)
            return 0

        lax.fori_loop(0, KD // 16, eb, 0)
    pltpu.sync_copy(acc, out_hbm.at[wid])


def _deg_call(dstp):
    return pl.kernel(
        _deg_body,
        out_type=jax.ShapeDtypeStruct((32, NP), _f32),
        scratch_types=[
            pltpu.VMEM((NP,), _f32),
            pltpu.VMEM((KD,), _i32),
            pltpu.VMEM((KD,), _i32),
            pltpu.SemaphoreType.DMA,
            pltpu.SemaphoreType.DMA,
        ],
        **_kw(),
    )(dstp)


def _step_body(ucols_hbm, src_hbm, dst_hbm, out_hbm,
               ucol, acc, sb0, db0, sb1, db1, ss0, sd0, ss1, sd1):
    wid = _wid()
    col = lax.rem(wid, 4)
    rng = wid // 4
    pltpu.sync_copy(ucols_hbm.at[col], ucol)
    _zero(acc, NP)
    base = rng * EPR
    nch = EPR // K
    sbufs = (sb0, sb1)
    dbufs = (db0, db1)
    ssems = (ss0, ss1)
    dsems = (sd0, sd1)

    def fire(ch):
        b = ch % 2
        pltpu.async_copy(src_hbm.at[pl.ds(base + ch * K, K)],
                         sbufs[b], ssems[b])
        pltpu.async_copy(dst_hbm.at[pl.ds(base + ch * K, K)],
                         dbufs[b], dsems[b])

    fire(0)
    for ch in range(nch):
        b = ch % 2
        pltpu.make_async_copy(src_hbm.at[pl.ds(base + ch * K, K)],
                              sbufs[b], ssems[b]).wait()
        pltpu.make_async_copy(dst_hbm.at[pl.ds(base + ch * K, K)],
                              dbufs[b], dsems[b]).wait()
        if ch + 1 < nch:
            fire(ch + 1)
        sbuf, dbuf = sbufs[b], dbufs[b]

        def eb(i, _):
            sl = pl.ds(i * 16, 16)
            g = plsc.load_gather(ucol, [sbuf[sl]])
            plsc.addupdate_scatter(acc, [dbuf[sl]], g)
            return 0

        lax.fori_loop(0, K // 16, eb, 0)
    pltpu.sync_copy(acc, out_hbm.at[rng, col])


def _step_call(ucols, srcp, dstp):
    return pl.kernel(
        _step_body,
        out_type=jax.ShapeDtypeStruct((8, 4, NP), _f32),
        scratch_types=[
            pltpu.VMEM((NP,), _f32),
            pltpu.VMEM((NP,), _f32),
            pltpu.VMEM((K,), _i32),
            pltpu.VMEM((K,), _i32),
            pltpu.VMEM((K,), _i32),
            pltpu.VMEM((K,), _i32),
            pltpu.SemaphoreType.DMA,
            pltpu.SemaphoreType.DMA,
            pltpu.SemaphoreType.DMA,
            pltpu.SemaphoreType.DMA,
        ],
        **_kw(),
    )(ucols, srcp, dstp)


_MR = NP // 8


def _merge_body(parts_hbm, ucols_hbm, sm_hbm, d1_hbm, un_hbm, zc_hbm,
                pbuf, ubuf, sbuf, dbuf, sumb, obuf, zbuf):
    wid = _wid()
    col = lax.rem(wid, 4)
    nrng = wid // 4
    off = nrng * _MR
    for r in range(8):
        pltpu.sync_copy(parts_hbm.at[r, col, pl.ds(off, _MR)], pbuf.at[r])
    pltpu.sync_copy(ucols_hbm.at[col, pl.ds(off, _MR)], ubuf)
    pltpu.sync_copy(sm_hbm.at[pl.ds(off, _MR)], sbuf)

    def mb(i, _):
        sl = pl.ds(i * 16, 16)
        sm = ubuf[sl]
        for r in range(8):
            sm = sm + pbuf[r, sl]
        sumb[sl] = sm
        obuf[sl] = sbuf[sl] * sm
        return 0

    lax.fori_loop(0, _MR // 16, mb, 0)
    pltpu.sync_copy(obuf, un_hbm.at[col, pl.ds(off, _MR)])

    @pl.when(col == 3)
    def _():
        pltpu.sync_copy(d1_hbm.at[pl.ds(off, _MR)], dbuf)

        def zcb(i, _):
            sl = pl.ds(i * 16, 16)
            zbuf[sl] = dbuf[sl] * sumb[sl]
            return 0

        lax.fori_loop(0, _MR // 16, zcb, 0)
        pltpu.sync_copy(zbuf, zc_hbm.at[pl.ds(off, _MR)])


def _merge_call(parts, ucols, smain, d1):
    return pl.kernel(
        _merge_body,
        out_type=(jax.ShapeDtypeStruct((4, NP), _f32),
                  jax.ShapeDtypeStruct((NP,), _f32)),
        scratch_types=[
            pltpu.VMEM((8, _MR), _f32),
            pltpu.VMEM((_MR,), _f32),
            pltpu.VMEM((_MR,), _f32),
            pltpu.VMEM((_MR,), _f32),
            pltpu.VMEM((_MR,), _f32),
            pltpu.VMEM((_MR,), _f32),
            pltpu.VMEM((_MR,), _f32),
        ],
        **_kw(),
    )(parts, ucols, smain, d1)


_SR = NP // 32


def _seg_body(z3_hbm, zc1_hbm, zc2_hbm, bp_hbm, out_hbm,
              accf, bbuf, v0, v1, v2, v3, c1b, c2b):
    wid = _wid()
    _zero(accf, GP * GW)
    off = wid * _SR
    pltpu.sync_copy(bp_hbm.at[pl.ds(off, _SR)], bbuf)
    for k, vb in enumerate((v0, v1, v2, v3)):
        pltpu.sync_copy(z3_hbm.at[k, pl.ds(off, _SR)], vb)
    pltpu.sync_copy(zc1_hbm.at[pl.ds(off, _SR)], c1b)
    pltpu.sync_copy(zc2_hbm.at[pl.ds(off, _SR)], c2b)
    ones16 = jnp.full((16,), 1.0, _f32)

    def eb(i, _):
        sl = pl.ds(i * 16, 16)
        ix = bbuf[sl] * GW
        for cst, vb in ((0, v0), (1, v1), (2, v2), (3, v3),
                        (4, c1b), (5, c2b)):
            plsc.addupdate_scatter(accf, [ix + cst], vb[sl])
        plsc.addupdate_scatter(accf, [ix + 6], ones16)
        return 0

    lax.fori_loop(0, _SR // 16, eb, 0)
    pltpu.sync_copy(accf, out_hbm.at[wid])


def _seg_call(z3c, zc1, zc2, bp):
    return pl.kernel(
        _seg_body,
        out_type=jax.ShapeDtypeStruct((32, GP * GW), _f32),
        scratch_types=[
            pltpu.VMEM((GP * GW,), _f32),
            pltpu.VMEM((_SR,), _i32),
            pltpu.VMEM((_SR,), _f32),
            pltpu.VMEM((_SR,), _f32),
            pltpu.VMEM((_SR,), _f32),
            pltpu.VMEM((_SR,), _f32),
            pltpu.VMEM((_SR,), _f32),
            pltpu.VMEM((_SR,), _f32),
        ],
        **_kw(),
    )(z3c, zc1, zc2, bp)


_RB = NP // 8


def _prep_body(degp_ref, xt_ref, u1_ref, d1_ref, d2_ref):
    deg = jnp.sum(degp_ref[...], axis=0, keepdims=True) + 1.0
    dinv = lax.rsqrt(deg)
    dinv = dinv * (1.5 - 0.5 * deg * dinv * dinv)
    ones = jnp.ones_like(deg)
    u1_ref[...] = jnp.concatenate([xt_ref[...], ones], axis=0) * dinv
    d1_ref[...] = dinv
    d2_ref[...] = dinv * dinv


def _prep_call(degp, xt):
    return pl.pallas_call(
        _prep_body,
        grid=(NP // _RB,),
        in_specs=[
            pl.BlockSpec((32, _RB), lambda i: (0, i)),
            pl.BlockSpec((3, _RB), lambda i: (0, i)),
        ],
        out_specs=[
            pl.BlockSpec((4, _RB), lambda i: (0, i)),
            pl.BlockSpec((1, _RB), lambda i: (0, i)),
            pl.BlockSpec((1, _RB), lambda i: (0, i)),
        ],
        out_shape=[
            jax.ShapeDtypeStruct((4, NP), _f32),
            jax.ShapeDtypeStruct((1, NP), _f32),
            jax.ShapeDtypeStruct((1, NP), _f32),
        ],
    )(degp, xt)


def _final_body(segp_ref, w1_ref, b1_ref, w2_ref, b2_ref,
                w3_ref, b3_ref, out_ref):
    s = jnp.sum(segp_ref[...], axis=0)
    m3 = s[:512, 0:3]
    z1s = s[:512, 4:5]
    z2s = s[:512, 5:6]
    cnt = jnp.maximum(s[:512, 6:7], 1.0)
    w12 = jnp.dot(w1_ref[...], w2_ref[...], preferred_element_type=_f32, precision=lax.Precision.HIGHEST)
    w123 = jnp.dot(w12, w3_ref[...], preferred_element_type=_f32, precision=lax.Precision.HIGHEST)
    v1 = jnp.dot(jnp.dot(b1_ref[...], w2_ref[...],
                         preferred_element_type=_f32, precision=lax.Precision.HIGHEST),
                 w3_ref[...], preferred_element_type=_f32, precision=lax.Precision.HIGHEST)
    v2 = jnp.dot(b2_ref[...], w3_ref[...], preferred_element_type=_f32, precision=lax.Precision.HIGHEST)
    num = jnp.dot(m3, w123, preferred_element_type=_f32, precision=lax.Precision.HIGHEST)
    num = num + z2s * v1 + z1s * v2
    out_ref[...] = num / cnt + b3_ref[...]


def _final_call(segp, W1, b1, W2, b2, W3, b3):
    return pl.pallas_call(
        _final_body,
        out_shape=jax.ShapeDtypeStruct((G, 192), _f32),
    )(segp, W1, b1, W2, b2, W3, b3)


def kernel(node_features, edge_indices, batch_indices,
           W1, b1, W2, b2, W3, b3):
    epad = jnp.full((EP - E,), N, _i32)
    srcp = jnp.concatenate([edge_indices[0], epad])
    dstp = jnp.concatenate([edge_indices[1], epad])
    xt = jnp.pad(node_features, ((0, NP - N), (0, 0))).T
    bp = jnp.pad(batch_indices, (0, NP - N), constant_values=G)

    degp = _deg_call(dstp)
    u1c, d1r, d2r = _prep_call(degp, xt)
    d1 = d1r.reshape(NP)
    d2 = d2r.reshape(NP)

    uc = u1c
    zcs = []
    for t in range(3):
        parts = _step_call(uc, srcp, dstp)
        smain = d2 if t < 2 else d1
        uc, zc = _merge_call(parts, uc, smain, d1)
        zcs.append(zc)

    segp = _seg_call(uc, zcs[0], zcs[1], bp).reshape(32, GP, GW)
    out = _final_call(segp, W1, b1.reshape(1, -1), W2,
                      b2.reshape(1, -1), W3, b3.reshape(1, -1))
    return out


# speedup vs baseline: 88.8982x; 1.3379x over previous
import jax
import jax.numpy as jnp
from jax import lax
from jax.experimental import pallas as pl
from jax.experimental.pallas import tpu as pltpu
from jax.experimental.pallas import tpu_sc as plsc

N = 50000
E = 800000
G = 512
NP = 50176
EP = 819200
EPR = EP // 8
EPW = EP // 32
K = 2048
KD = 2560
GP = 520
GW = 8

_f32 = jnp.float32
_i32 = jnp.int32

_SC_PARAMS = pltpu.CompilerParams(use_tc_tiling_on_sc=False,
                                  needs_layout_passes=False)


def _mesh():
    return plsc.VectorSubcoreMesh(core_axis_name="c", subcore_axis_name="s")


def _kw():
    return dict(mesh=_mesh(), compiler_params=_SC_PARAMS)


def _wid():
    return lax.axis_index("c") * 16 + lax.axis_index("s")


def _zero(buf, nwords):
    z = jnp.zeros((16,), _f32)

    @plsc.parallel_loop(0, nwords // 16, unroll=8)
    def _(i):
        buf[pl.ds(i * 16, 16)] = z


def _deg_body(dst_hbm, out_hbm, acc, b0, b1, s0, s1):
    wid = _wid()
    _zero(acc, NP)
    ones16 = jnp.full((16,), 1.0, _f32)
    base = wid * EPW
    nch = EPW // KD
    bufs = (b0, b1)
    sems = (s0, s1)

    def fire(ch):
        pltpu.async_copy(dst_hbm.at[pl.ds(base + ch * KD, KD)],
                         bufs[ch % 2], sems[ch % 2])

    fire(0)
    for ch in range(nch):
        buf, sem = bufs[ch % 2], sems[ch % 2]
        pltpu.make_async_copy(dst_hbm.at[pl.ds(base + ch * KD, KD)],
                              buf, sem).wait()
        if ch + 1 < nch:
            fire(ch + 1)

        @plsc.parallel_loop(0, KD // 16, unroll=8)
        def _(i):
            plsc.addupdate_scatter(acc, [buf[pl.ds(i * 16, 16)]], ones16)
    pltpu.sync_copy(acc, out_hbm.at[wid])


def _deg_call(dstp):
    return pl.kernel(
        _deg_body,
        out_type=jax.ShapeDtypeStruct((32, NP), _f32),
        scratch_types=[
            pltpu.VMEM((NP,), _f32),
            pltpu.VMEM((KD,), _i32),
            pltpu.VMEM((KD,), _i32),
            pltpu.SemaphoreType.DMA,
            pltpu.SemaphoreType.DMA,
        ],
        **_kw(),
    )(dstp)


def _step_body(ucols_hbm, src_hbm, dst_hbm, out_hbm,
               ucol, acc, sb0, db0, sb1, db1, ss0, sd0, ss1, sd1):
    wid = _wid()
    col = lax.rem(wid, 4)
    rng = wid // 4
    pltpu.sync_copy(ucols_hbm.at[col], ucol)
    _zero(acc, NP)
    base = rng * EPR
    nch = EPR // K
    sbufs = (sb0, sb1)
    dbufs = (db0, db1)
    ssems = (ss0, ss1)
    dsems = (sd0, sd1)

    def fire(ch):
        b = ch % 2
        pltpu.async_copy(src_hbm.at[pl.ds(base + ch * K, K)],
                         sbufs[b], ssems[b])
        pltpu.async_copy(dst_hbm.at[pl.ds(base + ch * K, K)],
                         dbufs[b], dsems[b])

    fire(0)
    for ch in range(nch):
        b = ch % 2
        pltpu.make_async_copy(src_hbm.at[pl.ds(base + ch * K, K)],
                              sbufs[b], ssems[b]).wait()
        pltpu.make_async_copy(dst_hbm.at[pl.ds(base + ch * K, K)],
                              dbufs[b], dsems[b]).wait()
        if ch + 1 < nch:
            fire(ch + 1)
        sbuf, dbuf = sbufs[b], dbufs[b]

        @plsc.parallel_loop(0, K // 16, unroll=8)
        def _(i):
            sl = pl.ds(i * 16, 16)
            g = plsc.load_gather(ucol, [sbuf[sl]])
            plsc.addupdate_scatter(acc, [dbuf[sl]], g)
    pltpu.sync_copy(acc, out_hbm.at[rng, col])


def _step_call(ucols, srcp, dstp):
    return pl.kernel(
        _step_body,
        out_type=jax.ShapeDtypeStruct((8, 4, NP), _f32),
        scratch_types=[
            pltpu.VMEM((NP,), _f32),
            pltpu.VMEM((NP,), _f32),
            pltpu.VMEM((K,), _i32),
            pltpu.VMEM((K,), _i32),
            pltpu.VMEM((K,), _i32),
            pltpu.VMEM((K,), _i32),
            pltpu.SemaphoreType.DMA,
            pltpu.SemaphoreType.DMA,
            pltpu.SemaphoreType.DMA,
            pltpu.SemaphoreType.DMA,
        ],
        **_kw(),
    )(ucols, srcp, dstp)


_MR = NP // 8


def _merge_body(parts_hbm, ucols_hbm, sm_hbm, d1_hbm, un_hbm, zc_hbm,
                pbuf, ubuf, sbuf, dbuf, sumb, obuf, zbuf):
    wid = _wid()
    col = lax.rem(wid, 4)
    nrng = wid // 4
    off = nrng * _MR
    for r in range(8):
        pltpu.sync_copy(parts_hbm.at[r, col, pl.ds(off, _MR)], pbuf.at[r])
    pltpu.sync_copy(ucols_hbm.at[col, pl.ds(off, _MR)], ubuf)
    pltpu.sync_copy(sm_hbm.at[pl.ds(off, _MR)], sbuf)

    @plsc.parallel_loop(0, _MR // 16, unroll=4)
    def _(i):
        sl = pl.ds(i * 16, 16)
        sm = ubuf[sl]
        for r in range(8):
            sm = sm + pbuf[r, sl]
        sumb[sl] = sm
        obuf[sl] = sbuf[sl] * sm
    pltpu.sync_copy(obuf, un_hbm.at[col, pl.ds(off, _MR)])

    @pl.when(col == 3)
    def _():
        pltpu.sync_copy(d1_hbm.at[pl.ds(off, _MR)], dbuf)

        @plsc.parallel_loop(0, _MR // 16, unroll=8)
        def _(i):
            sl = pl.ds(i * 16, 16)
            zbuf[sl] = dbuf[sl] * sumb[sl]
        pltpu.sync_copy(zbuf, zc_hbm.at[pl.ds(off, _MR)])


def _merge_call(parts, ucols, smain, d1):
    return pl.kernel(
        _merge_body,
        out_type=(jax.ShapeDtypeStruct((4, NP), _f32),
                  jax.ShapeDtypeStruct((NP,), _f32)),
        scratch_types=[
            pltpu.VMEM((8, _MR), _f32),
            pltpu.VMEM((_MR,), _f32),
            pltpu.VMEM((_MR,), _f32),
            pltpu.VMEM((_MR,), _f32),
            pltpu.VMEM((_MR,), _f32),
            pltpu.VMEM((_MR,), _f32),
            pltpu.VMEM((_MR,), _f32),
        ],
        **_kw(),
    )(parts, ucols, smain, d1)


_SR = NP // 32


def _seg_body(z3_hbm, zc1_hbm, zc2_hbm, bp_hbm, out_hbm,
              accf, bbuf, v0, v1, v2, v3, c1b, c2b):
    wid = _wid()
    _zero(accf, GP * GW)
    off = wid * _SR
    pltpu.sync_copy(bp_hbm.at[pl.ds(off, _SR)], bbuf)
    for k, vb in enumerate((v0, v1, v2, v3)):
        pltpu.sync_copy(z3_hbm.at[k, pl.ds(off, _SR)], vb)
    pltpu.sync_copy(zc1_hbm.at[pl.ds(off, _SR)], c1b)
    pltpu.sync_copy(zc2_hbm.at[pl.ds(off, _SR)], c2b)
    ones16 = jnp.full((16,), 1.0, _f32)

    @plsc.parallel_loop(0, _SR // 16, unroll=2)
    def _(i):
        sl = pl.ds(i * 16, 16)
        ix = bbuf[sl] * GW
        for cst, vb in ((0, v0), (1, v1), (2, v2), (3, v3),
                        (4, c1b), (5, c2b)):
            plsc.addupdate_scatter(accf, [ix + cst], vb[sl])
        plsc.addupdate_scatter(accf, [ix + 6], ones16)
    pltpu.sync_copy(accf, out_hbm.at[wid])


def _seg_call(z3c, zc1, zc2, bp):
    return pl.kernel(
        _seg_body,
        out_type=jax.ShapeDtypeStruct((32, GP * GW), _f32),
        scratch_types=[
            pltpu.VMEM((GP * GW,), _f32),
            pltpu.VMEM((_SR,), _i32),
            pltpu.VMEM((_SR,), _f32),
            pltpu.VMEM((_SR,), _f32),
            pltpu.VMEM((_SR,), _f32),
            pltpu.VMEM((_SR,), _f32),
            pltpu.VMEM((_SR,), _f32),
            pltpu.VMEM((_SR,), _f32),
        ],
        **_kw(),
    )(z3c, zc1, zc2, bp)


_RB = NP // 8


def _prep_body(degp_ref, xt_ref, u1_ref, d1_ref, d2_ref):
    deg = jnp.sum(degp_ref[...], axis=0, keepdims=True) + 1.0
    dinv = lax.rsqrt(deg)
    dinv = dinv * (1.5 - 0.5 * deg * dinv * dinv)
    ones = jnp.ones_like(deg)
    u1_ref[...] = jnp.concatenate([xt_ref[...], ones], axis=0) * dinv
    d1_ref[...] = dinv
    d2_ref[...] = dinv * dinv


def _prep_call(degp, xt):
    return pl.pallas_call(
        _prep_body,
        grid=(NP // _RB,),
        in_specs=[
            pl.BlockSpec((32, _RB), lambda i: (0, i)),
            pl.BlockSpec((3, _RB), lambda i: (0, i)),
        ],
        out_specs=[
            pl.BlockSpec((4, _RB), lambda i: (0, i)),
            pl.BlockSpec((1, _RB), lambda i: (0, i)),
            pl.BlockSpec((1, _RB), lambda i: (0, i)),
        ],
        out_shape=[
            jax.ShapeDtypeStruct((4, NP), _f32),
            jax.ShapeDtypeStruct((1, NP), _f32),
            jax.ShapeDtypeStruct((1, NP), _f32),
        ],
    )(degp, xt)


def _final_body(segp_ref, w1_ref, b1_ref, w2_ref, b2_ref,
                w3_ref, b3_ref, out_ref):
    s = jnp.sum(segp_ref[...], axis=0)
    m3 = s[:512, 0:3]
    z1s = s[:512, 4:5]
    z2s = s[:512, 5:6]
    cnt = jnp.maximum(s[:512, 6:7], 1.0)
    w12 = jnp.dot(w1_ref[...], w2_ref[...], preferred_element_type=_f32, precision=lax.Precision.HIGHEST)
    w123 = jnp.dot(w12, w3_ref[...], preferred_element_type=_f32, precision=lax.Precision.HIGHEST)
    v1 = jnp.dot(jnp.dot(b1_ref[...], w2_ref[...],
                         preferred_element_type=_f32, precision=lax.Precision.HIGHEST),
                 w3_ref[...], preferred_element_type=_f32, precision=lax.Precision.HIGHEST)
    v2 = jnp.dot(b2_ref[...], w3_ref[...], preferred_element_type=_f32, precision=lax.Precision.HIGHEST)
    num = jnp.dot(m3, w123, preferred_element_type=_f32, precision=lax.Precision.HIGHEST)
    num = num + z2s * v1 + z1s * v2
    out_ref[...] = num / cnt + b3_ref[...]


def _final_call(segp, W1, b1, W2, b2, W3, b3):
    return pl.pallas_call(
        _final_body,
        out_shape=jax.ShapeDtypeStruct((G, 192), _f32),
    )(segp, W1, b1, W2, b2, W3, b3)


def kernel(node_features, edge_indices, batch_indices,
           W1, b1, W2, b2, W3, b3):
    epad = jnp.full((EP - E,), N, _i32)
    srcp = jnp.concatenate([edge_indices[0], epad])
    dstp = jnp.concatenate([edge_indices[1], epad])
    xt = jnp.pad(node_features, ((0, NP - N), (0, 0))).T
    bp = jnp.pad(batch_indices, (0, NP - N), constant_values=G)

    degp = _deg_call(dstp)
    u1c, d1r, d2r = _prep_call(degp, xt)
    d1 = d1r.reshape(NP)
    d2 = d2r.reshape(NP)

    uc = u1c
    zcs = []
    for t in range(3):
        parts = _step_call(uc, srcp, dstp)
        smain = d2 if t < 2 else d1
        uc, zc = _merge_call(parts, uc, smain, d1)
        zcs.append(zc)

    segp = _seg_call(uc, zcs[0], zcs[1], bp).reshape(32, GP, GW)
    out = _final_call(segp, W1, b1.reshape(1, -1), W2,
                      b2.reshape(1, -1), W3, b3.reshape(1, -1))
    return out


# speedup vs baseline: 96.2722x; 1.0829x over previous
import jax
import jax.numpy as jnp
from jax import lax
from jax.experimental import pallas as pl
from jax.experimental.pallas import tpu as pltpu
from jax.experimental.pallas import tpu_sc as plsc

N = 50000
E = 800000
G = 512
NP = 50176
EP = 819200
EPR = EP // 8
EPW = EP // 32
K = 4096
KD = 5120
GP = 520
GW = 8

_f32 = jnp.float32
_i32 = jnp.int32

_SC_PARAMS = pltpu.CompilerParams(use_tc_tiling_on_sc=False,
                                  needs_layout_passes=False)


def _mesh():
    return plsc.VectorSubcoreMesh(core_axis_name="c", subcore_axis_name="s")


def _kw():
    return dict(mesh=_mesh(), compiler_params=_SC_PARAMS)


def _wid():
    return lax.axis_index("c") * 16 + lax.axis_index("s")


def _zero(buf, nwords):
    z = jnp.zeros((16,), _f32)

    @plsc.parallel_loop(0, nwords // 16, unroll=8)
    def _(i):
        buf[pl.ds(i * 16, 16)] = z


def _deg_body(dst_hbm, out_hbm, acc, b0, b1, s0, s1):
    wid = _wid()
    _zero(acc, NP)
    ones16 = jnp.full((16,), 1.0, _f32)
    base = wid * EPW
    nch = EPW // KD
    bufs = (b0, b1)
    sems = (s0, s1)

    def fire(ch):
        pltpu.async_copy(dst_hbm.at[pl.ds(base + ch * KD, KD)],
                         bufs[ch % 2], sems[ch % 2])

    fire(0)
    for ch in range(nch):
        buf, sem = bufs[ch % 2], sems[ch % 2]
        pltpu.make_async_copy(dst_hbm.at[pl.ds(base + ch * KD, KD)],
                              buf, sem).wait()
        if ch + 1 < nch:
            fire(ch + 1)

        @plsc.parallel_loop(0, KD // 16, unroll=16)
        def _(i):
            plsc.addupdate_scatter(acc, [buf[pl.ds(i * 16, 16)]], ones16)
    pltpu.sync_copy(acc, out_hbm.at[wid])


def _deg_call(dstp):
    return pl.kernel(
        _deg_body,
        out_type=jax.ShapeDtypeStruct((32, NP), _f32),
        scratch_types=[
            pltpu.VMEM((NP,), _f32),
            pltpu.VMEM((KD,), _i32),
            pltpu.VMEM((KD,), _i32),
            pltpu.SemaphoreType.DMA,
            pltpu.SemaphoreType.DMA,
        ],
        **_kw(),
    )(dstp)


def _step_body(ucols_hbm, src_hbm, dst_hbm, out_hbm,
               ucol, acc, sb0, db0, sb1, db1, ss0, sd0, ss1, sd1):
    wid = _wid()
    col = lax.rem(wid, 4)
    rng = wid // 4
    pltpu.sync_copy(ucols_hbm.at[col], ucol)
    _zero(acc, NP)
    base = rng * EPR
    nch = EPR // K
    sbufs = (sb0, sb1)
    dbufs = (db0, db1)
    ssems = (ss0, ss1)
    dsems = (sd0, sd1)

    def fire(ch):
        b = ch % 2
        pltpu.async_copy(src_hbm.at[pl.ds(base + ch * K, K)],
                         sbufs[b], ssems[b])
        pltpu.async_copy(dst_hbm.at[pl.ds(base + ch * K, K)],
                         dbufs[b], dsems[b])

    fire(0)
    for ch in range(nch):
        b = ch % 2
        pltpu.make_async_copy(src_hbm.at[pl.ds(base + ch * K, K)],
                              sbufs[b], ssems[b]).wait()
        pltpu.make_async_copy(dst_hbm.at[pl.ds(base + ch * K, K)],
                              dbufs[b], dsems[b]).wait()
        if ch + 1 < nch:
            fire(ch + 1)
        sbuf, dbuf = sbufs[b], dbufs[b]

        @plsc.parallel_loop(0, K // 16, unroll=16)
        def _(i):
            sl = pl.ds(i * 16, 16)
            g = plsc.load_gather(ucol, [sbuf[sl]])
            plsc.addupdate_scatter(acc, [dbuf[sl]], g)
    pltpu.sync_copy(acc, out_hbm.at[rng, col])


def _step_call(ucols, srcp, dstp):
    return pl.kernel(
        _step_body,
        out_type=jax.ShapeDtypeStruct((8, 4, NP), _f32),
        scratch_types=[
            pltpu.VMEM((NP,), _f32),
            pltpu.VMEM((NP,), _f32),
            pltpu.VMEM((K,), _i32),
            pltpu.VMEM((K,), _i32),
            pltpu.VMEM((K,), _i32),
            pltpu.VMEM((K,), _i32),
            pltpu.SemaphoreType.DMA,
            pltpu.SemaphoreType.DMA,
            pltpu.SemaphoreType.DMA,
            pltpu.SemaphoreType.DMA,
        ],
        **_kw(),
    )(ucols, srcp, dstp)


_MR = NP // 8


def _merge_body(parts_hbm, ucols_hbm, sm_hbm, d1_hbm, un_hbm, zc_hbm,
                pbuf, ubuf, sbuf, dbuf, sumb, obuf, zbuf):
    wid = _wid()
    col = lax.rem(wid, 4)
    nrng = wid // 4
    off = nrng * _MR
    for r in range(8):
        pltpu.sync_copy(parts_hbm.at[r, col, pl.ds(off, _MR)], pbuf.at[r])
    pltpu.sync_copy(ucols_hbm.at[col, pl.ds(off, _MR)], ubuf)
    pltpu.sync_copy(sm_hbm.at[pl.ds(off, _MR)], sbuf)

    @plsc.parallel_loop(0, _MR // 16, unroll=4)
    def _(i):
        sl = pl.ds(i * 16, 16)
        sm = ubuf[sl]
        for r in range(8):
            sm = sm + pbuf[r, sl]
        sumb[sl] = sm
        obuf[sl] = sbuf[sl] * sm
    pltpu.sync_copy(obuf, un_hbm.at[col, pl.ds(off, _MR)])

    @pl.when(col == 3)
    def _():
        pltpu.sync_copy(d1_hbm.at[pl.ds(off, _MR)], dbuf)

        @plsc.parallel_loop(0, _MR // 16, unroll=8)
        def _(i):
            sl = pl.ds(i * 16, 16)
            zbuf[sl] = dbuf[sl] * sumb[sl]
        pltpu.sync_copy(zbuf, zc_hbm.at[pl.ds(off, _MR)])


def _merge_call(parts, ucols, smain, d1):
    return pl.kernel(
        _merge_body,
        out_type=(jax.ShapeDtypeStruct((4, NP), _f32),
                  jax.ShapeDtypeStruct((NP,), _f32)),
        scratch_types=[
            pltpu.VMEM((8, _MR), _f32),
            pltpu.VMEM((_MR,), _f32),
            pltpu.VMEM((_MR,), _f32),
            pltpu.VMEM((_MR,), _f32),
            pltpu.VMEM((_MR,), _f32),
            pltpu.VMEM((_MR,), _f32),
            pltpu.VMEM((_MR,), _f32),
        ],
        **_kw(),
    )(parts, ucols, smain, d1)


_SR = NP // 32


def _seg_body(z3_hbm, zc1_hbm, zc2_hbm, bp_hbm, out_hbm,
              accf, bbuf, v0, v1, v2, v3, c1b, c2b):
    wid = _wid()
    _zero(accf, GP * GW)
    off = wid * _SR
    pltpu.sync_copy(bp_hbm.at[pl.ds(off, _SR)], bbuf)
    for k, vb in enumerate((v0, v1, v2, v3)):
        pltpu.sync_copy(z3_hbm.at[k, pl.ds(off, _SR)], vb)
    pltpu.sync_copy(zc1_hbm.at[pl.ds(off, _SR)], c1b)
    pltpu.sync_copy(zc2_hbm.at[pl.ds(off, _SR)], c2b)
    ones16 = jnp.full((16,), 1.0, _f32)

    @plsc.parallel_loop(0, _SR // 16, unroll=2)
    def _(i):
        sl = pl.ds(i * 16, 16)
        ix = bbuf[sl] * GW
        for cst, vb in ((0, v0), (1, v1), (2, v2), (3, v3),
                        (4, c1b), (5, c2b)):
            plsc.addupdate_scatter(accf, [ix + cst], vb[sl])
        plsc.addupdate_scatter(accf, [ix + 6], ones16)
    pltpu.sync_copy(accf, out_hbm.at[wid])


def _seg_call(z3c, zc1, zc2, bp):
    return pl.kernel(
        _seg_body,
        out_type=jax.ShapeDtypeStruct((32, GP * GW), _f32),
        scratch_types=[
            pltpu.VMEM((GP * GW,), _f32),
            pltpu.VMEM((_SR,), _i32),
            pltpu.VMEM((_SR,), _f32),
            pltpu.VMEM((_SR,), _f32),
            pltpu.VMEM((_SR,), _f32),
            pltpu.VMEM((_SR,), _f32),
            pltpu.VMEM((_SR,), _f32),
            pltpu.VMEM((_SR,), _f32),
        ],
        **_kw(),
    )(z3c, zc1, zc2, bp)


_RB = NP // 8


def _prep_body(degp_ref, xt_ref, u1_ref, d1_ref, d2_ref):
    deg = jnp.sum(degp_ref[...], axis=0, keepdims=True) + 1.0
    dinv = lax.rsqrt(deg)
    dinv = dinv * (1.5 - 0.5 * deg * dinv * dinv)
    ones = jnp.ones_like(deg)
    u1_ref[...] = jnp.concatenate([xt_ref[...], ones], axis=0) * dinv
    d1_ref[...] = dinv
    d2_ref[...] = dinv * dinv


def _prep_call(degp, xt):
    return pl.pallas_call(
        _prep_body,
        grid=(NP // _RB,),
        in_specs=[
            pl.BlockSpec((32, _RB), lambda i: (0, i)),
            pl.BlockSpec((3, _RB), lambda i: (0, i)),
        ],
        out_specs=[
            pl.BlockSpec((4, _RB), lambda i: (0, i)),
            pl.BlockSpec((1, _RB), lambda i: (0, i)),
            pl.BlockSpec((1, _RB), lambda i: (0, i)),
        ],
        out_shape=[
            jax.ShapeDtypeStruct((4, NP), _f32),
            jax.ShapeDtypeStruct((1, NP), _f32),
            jax.ShapeDtypeStruct((1, NP), _f32),
        ],
    )(degp, xt)


def _final_body(segp_ref, w1_ref, b1_ref, w2_ref, b2_ref,
                w3_ref, b3_ref, out_ref):
    s = jnp.sum(segp_ref[...], axis=0)
    m3 = s[:512, 0:3]
    z1s = s[:512, 4:5]
    z2s = s[:512, 5:6]
    cnt = jnp.maximum(s[:512, 6:7], 1.0)
    w12 = jnp.dot(w1_ref[...], w2_ref[...], preferred_element_type=_f32, precision=lax.Precision.HIGHEST)
    w123 = jnp.dot(w12, w3_ref[...], preferred_element_type=_f32, precision=lax.Precision.HIGHEST)
    v1 = jnp.dot(jnp.dot(b1_ref[...], w2_ref[...],
                         preferred_element_type=_f32, precision=lax.Precision.HIGHEST),
                 w3_ref[...], preferred_element_type=_f32, precision=lax.Precision.HIGHEST)
    v2 = jnp.dot(b2_ref[...], w3_ref[...], preferred_element_type=_f32, precision=lax.Precision.HIGHEST)
    num = jnp.dot(m3, w123, preferred_element_type=_f32, precision=lax.Precision.HIGHEST)
    num = num + z2s * v1 + z1s * v2
    out_ref[...] = num / cnt + b3_ref[...]


def _final_call(segp, W1, b1, W2, b2, W3, b3):
    return pl.pallas_call(
        _final_body,
        out_shape=jax.ShapeDtypeStruct((G, 192), _f32),
    )(segp, W1, b1, W2, b2, W3, b3)


def kernel(node_features, edge_indices, batch_indices,
           W1, b1, W2, b2, W3, b3):
    epad = jnp.full((EP - E,), N, _i32)
    srcp = jnp.concatenate([edge_indices[0], epad])
    dstp = jnp.concatenate([edge_indices[1], epad])
    xt = jnp.pad(node_features, ((0, NP - N), (0, 0))).T
    bp = jnp.pad(batch_indices, (0, NP - N), constant_values=G)

    degp = _deg_call(dstp)
    u1c, d1r, d2r = _prep_call(degp, xt)
    d1 = d1r.reshape(NP)
    d2 = d2r.reshape(NP)

    uc = u1c
    zcs = []
    for t in range(3):
        parts = _step_call(uc, srcp, dstp)
        smain = d2 if t < 2 else d1
        uc, zc = _merge_call(parts, uc, smain, d1)
        zcs.append(zc)

    segp = _seg_call(uc, zcs[0], zcs[1], bp).reshape(32, GP, GW)
    out = _final_call(segp, W1, b1.reshape(1, -1), W2,
                      b2.reshape(1, -1), W3, b3.reshape(1, -1))
    return out


# speedup vs baseline: 120.0552x; 1.2470x over previous
import jax
import jax.numpy as jnp
from jax import lax
from jax.experimental import pallas as pl
from jax.experimental.pallas import tpu as pltpu
from jax.experimental.pallas import tpu_sc as plsc

N = 50000
E = 800000
G = 512
NP = 50176
EP = 819200
EPR = EP // 8
EPW = EP // 32
K = 4096
KD = 5120
GP = 520
GW = 8

_f32 = jnp.float32
_i32 = jnp.int32

_SC_PARAMS = pltpu.CompilerParams(use_tc_tiling_on_sc=False,
                                  needs_layout_passes=False)


def _mesh():
    return plsc.VectorSubcoreMesh(core_axis_name="c", subcore_axis_name="s")


def _kw():
    return dict(mesh=_mesh(), compiler_params=_SC_PARAMS)


def _wid():
    return lax.axis_index("c") * 16 + lax.axis_index("s")


def _zero(buf, nwords):
    z = jnp.zeros((16,), _f32)

    @plsc.parallel_loop(0, nwords // 16, unroll=8)
    def _(i):
        buf[pl.ds(i * 16, 16)] = z


def _deg_body(dst_hbm, out_hbm, acc, b0, b1, s0, s1):
    wid = _wid()
    _zero(acc, NP)
    ones16 = jnp.full((16,), 1.0, _f32)
    base = wid * EPW
    nch = EPW // KD
    bufs = (b0, b1)
    sems = (s0, s1)

    def fire(ch):
        pltpu.async_copy(dst_hbm.at[pl.ds(base + ch * KD, KD)],
                         bufs[ch % 2], sems[ch % 2])

    fire(0)
    for ch in range(nch):
        buf, sem = bufs[ch % 2], sems[ch % 2]
        pltpu.make_async_copy(dst_hbm.at[pl.ds(base + ch * KD, KD)],
                              buf, sem).wait()
        if ch + 1 < nch:
            fire(ch + 1)

        @plsc.parallel_loop(0, KD // 16, unroll=16)
        def _(i):
            plsc.addupdate_scatter(acc, [buf[pl.ds(i * 16, 16)]], ones16)
    pltpu.sync_copy(acc, out_hbm.at[wid])


def _deg_call(dstp):
    return pl.kernel(
        _deg_body,
        out_type=jax.ShapeDtypeStruct((32, NP), _f32),
        scratch_types=[
            pltpu.VMEM((NP,), _f32),
            pltpu.VMEM((KD,), _i32),
            pltpu.VMEM((KD,), _i32),
            pltpu.SemaphoreType.DMA,
            pltpu.SemaphoreType.DMA,
        ],
        **_kw(),
    )(dstp)


def _step_body(ucols_hbm, src_hbm, dst_hbm, out_hbm,
               ucol, acc, sb0, db0, sb1, db1, ss0, sd0, ss1, sd1):
    wid = _wid()
    col = lax.rem(wid, 4)
    rng = wid // 4
    pltpu.sync_copy(ucols_hbm.at[col], ucol)
    _zero(acc, NP)
    base = rng * EPR
    nch = EPR // K
    sbufs = (sb0, sb1)
    dbufs = (db0, db1)
    ssems = (ss0, ss1)
    dsems = (sd0, sd1)

    def fire(ch):
        b = ch % 2
        pltpu.async_copy(src_hbm.at[pl.ds(base + ch * K, K)],
                         sbufs[b], ssems[b])
        pltpu.async_copy(dst_hbm.at[pl.ds(base + ch * K, K)],
                         dbufs[b], dsems[b])

    fire(0)
    for ch in range(nch):
        b = ch % 2
        pltpu.make_async_copy(src_hbm.at[pl.ds(base + ch * K, K)],
                              sbufs[b], ssems[b]).wait()
        pltpu.make_async_copy(dst_hbm.at[pl.ds(base + ch * K, K)],
                              dbufs[b], dsems[b]).wait()
        if ch + 1 < nch:
            fire(ch + 1)
        sbuf, dbuf = sbufs[b], dbufs[b]

        @plsc.parallel_loop(0, K // 16, unroll=16)
        def _(i):
            sl = pl.ds(i * 16, 16)
            g = plsc.load_gather(ucol, [sbuf[sl]])
            plsc.addupdate_scatter(acc, [dbuf[sl]], g)
    pltpu.sync_copy(acc, out_hbm.at[rng, col])


def _step_call(ucols, srcp, dstp):
    return pl.kernel(
        _step_body,
        out_type=jax.ShapeDtypeStruct((8, 4, NP), _f32),
        scratch_types=[
            pltpu.VMEM((NP,), _f32),
            pltpu.VMEM((NP,), _f32),
            pltpu.VMEM((K,), _i32),
            pltpu.VMEM((K,), _i32),
            pltpu.VMEM((K,), _i32),
            pltpu.VMEM((K,), _i32),
            pltpu.SemaphoreType.DMA,
            pltpu.SemaphoreType.DMA,
            pltpu.SemaphoreType.DMA,
            pltpu.SemaphoreType.DMA,
        ],
        **_kw(),
    )(ucols, srcp, dstp)


_MR = NP // 8


def _merge_body(parts_hbm, ucols_hbm, sm_hbm, d1_hbm, un_hbm, zc_hbm,
                pbuf, ubuf, sbuf, dbuf, sumb, obuf, zbuf):
    wid = _wid()
    col = lax.rem(wid, 4)
    nrng = wid // 4
    off = nrng * _MR
    for r in range(8):
        pltpu.sync_copy(parts_hbm.at[r, col, pl.ds(off, _MR)], pbuf.at[r])
    pltpu.sync_copy(ucols_hbm.at[col, pl.ds(off, _MR)], ubuf)
    pltpu.sync_copy(sm_hbm.at[pl.ds(off, _MR)], sbuf)

    @plsc.parallel_loop(0, _MR // 16, unroll=4)
    def _(i):
        sl = pl.ds(i * 16, 16)
        sm = ubuf[sl]
        for r in range(8):
            sm = sm + pbuf[r, sl]
        sumb[sl] = sm
        obuf[sl] = sbuf[sl] * sm
    pltpu.sync_copy(obuf, un_hbm.at[col, pl.ds(off, _MR)])

    @pl.when(col == 3)
    def _():
        pltpu.sync_copy(d1_hbm.at[pl.ds(off, _MR)], dbuf)

        @plsc.parallel_loop(0, _MR // 16, unroll=8)
        def _(i):
            sl = pl.ds(i * 16, 16)
            zbuf[sl] = dbuf[sl] * sumb[sl]
        pltpu.sync_copy(zbuf, zc_hbm.at[pl.ds(off, _MR)])


def _merge_call(parts, ucols, smain, d1):
    return pl.kernel(
        _merge_body,
        out_type=(jax.ShapeDtypeStruct((4, NP), _f32),
                  jax.ShapeDtypeStruct((NP,), _f32)),
        scratch_types=[
            pltpu.VMEM((8, _MR), _f32),
            pltpu.VMEM((_MR,), _f32),
            pltpu.VMEM((_MR,), _f32),
            pltpu.VMEM((_MR,), _f32),
            pltpu.VMEM((_MR,), _f32),
            pltpu.VMEM((_MR,), _f32),
            pltpu.VMEM((_MR,), _f32),
        ],
        **_kw(),
    )(parts, ucols, smain, d1)


_SR = NP // 32


def _seg_body(z3_hbm, zc1_hbm, zc2_hbm, bp_hbm, out_hbm,
              accf, bbuf, v0, v1, v2, v3, c1b, c2b):
    wid = _wid()
    _zero(accf, GP * GW)
    off = wid * _SR
    pltpu.sync_copy(bp_hbm.at[pl.ds(off, _SR)], bbuf)
    for k, vb in enumerate((v0, v1, v2, v3)):
        pltpu.sync_copy(z3_hbm.at[k, pl.ds(off, _SR)], vb)
    pltpu.sync_copy(zc1_hbm.at[pl.ds(off, _SR)], c1b)
    pltpu.sync_copy(zc2_hbm.at[pl.ds(off, _SR)], c2b)
    ones16 = jnp.full((16,), 1.0, _f32)

    @plsc.parallel_loop(0, _SR // 16, unroll=2)
    def _(i):
        sl = pl.ds(i * 16, 16)
        ix = bbuf[sl] * GW
        for cst, vb in ((0, v0), (1, v1), (2, v2), (3, v3),
                        (4, c1b), (5, c2b)):
            plsc.addupdate_scatter(accf, [ix + cst], vb[sl])
        plsc.addupdate_scatter(accf, [ix + 6], ones16)
    pltpu.sync_copy(accf, out_hbm.at[wid])


def _seg_call(z3c, zc1, zc2, bp):
    return pl.kernel(
        _seg_body,
        out_type=jax.ShapeDtypeStruct((32, GP * GW), _f32),
        scratch_types=[
            pltpu.VMEM((GP * GW,), _f32),
            pltpu.VMEM((_SR,), _i32),
            pltpu.VMEM((_SR,), _f32),
            pltpu.VMEM((_SR,), _f32),
            pltpu.VMEM((_SR,), _f32),
            pltpu.VMEM((_SR,), _f32),
            pltpu.VMEM((_SR,), _f32),
            pltpu.VMEM((_SR,), _f32),
        ],
        **_kw(),
    )(z3c, zc1, zc2, bp)


_RB = NP // 8


def _prep_body(degp_ref, xt_ref, u1_ref, d1_ref, d2_ref):
    deg = jnp.sum(degp_ref[...], axis=0, keepdims=True) + 1.0
    dinv = lax.rsqrt(deg)
    dinv = dinv * (1.5 - 0.5 * deg * dinv * dinv)
    ones = jnp.ones_like(deg)
    u1_ref[...] = jnp.concatenate([xt_ref[...], ones], axis=0) * dinv
    d1_ref[...] = dinv
    d2_ref[...] = dinv * dinv


def _prep_call(degp, xt):
    return pl.pallas_call(
        _prep_body,
        grid=(NP // _RB,),
        in_specs=[
            pl.BlockSpec((32, _RB), lambda i: (0, i)),
            pl.BlockSpec((3, _RB), lambda i: (0, i)),
        ],
        out_specs=[
            pl.BlockSpec((4, _RB), lambda i: (0, i)),
            pl.BlockSpec((1, _RB), lambda i: (0, i)),
            pl.BlockSpec((1, _RB), lambda i: (0, i)),
        ],
        out_shape=[
            jax.ShapeDtypeStruct((4, NP), _f32),
            jax.ShapeDtypeStruct((1, NP), _f32),
            jax.ShapeDtypeStruct((1, NP), _f32),
        ],
    )(degp, xt)


def _final_body(segp_ref, w1_ref, b1_ref, w2_ref, b2_ref,
                w3_ref, b3_ref, out_ref):
    s = jnp.sum(segp_ref[...], axis=0)
    m3 = s[:512, 0:3]
    z1s = s[:512, 4:5]
    z2s = s[:512, 5:6]
    cnt = jnp.maximum(s[:512, 6:7], 1.0)
    w12 = jnp.dot(w1_ref[...], w2_ref[...], preferred_element_type=_f32, precision=lax.Precision.HIGHEST)
    w123 = jnp.dot(w12, w3_ref[...], preferred_element_type=_f32, precision=lax.Precision.HIGHEST)
    v1 = jnp.dot(jnp.dot(b1_ref[...], w2_ref[...],
                         preferred_element_type=_f32, precision=lax.Precision.HIGHEST),
                 w3_ref[...], preferred_element_type=_f32, precision=lax.Precision.HIGHEST)
    v2 = jnp.dot(b2_ref[...], w3_ref[...], preferred_element_type=_f32, precision=lax.Precision.HIGHEST)
    num = jnp.dot(m3, w123, preferred_element_type=_f32, precision=lax.Precision.HIGHEST)
    num = num + z2s * v1 + z1s * v2
    out_ref[...] = num / cnt + b3_ref[...]


def _final_call(segp, W1, b1, W2, b2, W3, b3):
    return pl.pallas_call(
        _final_body,
        out_shape=jax.ShapeDtypeStruct((G, 192), _f32),
    )(segp, W1, b1, W2, b2, W3, b3)


def kernel(node_features, edge_indices, batch_indices,
           W1, b1, W2, b2, W3, b3):
    epad = N + jnp.arange(EP - E, dtype=_i32) % (NP - N)
    srcp = jnp.concatenate([edge_indices[0], epad])
    dstp = jnp.concatenate([edge_indices[1], epad])
    xt = jnp.pad(node_features, ((0, NP - N), (0, 0))).T
    bp = jnp.pad(batch_indices, (0, NP - N), constant_values=G)

    degp = _deg_call(dstp)
    u1c, d1r, d2r = _prep_call(degp, xt)
    d1 = d1r.reshape(NP)
    d2 = d2r.reshape(NP)

    uc = u1c
    zcs = []
    for t in range(3):
        parts = _step_call(uc, srcp, dstp)
        smain = d2 if t < 2 else d1
        uc, zc = _merge_call(parts, uc, smain, d1)
        zcs.append(zc)

    segp = _seg_call(uc, zcs[0], zcs[1], bp).reshape(32, GP, GW)
    out = _final_call(segp, W1, b1.reshape(1, -1), W2,
                      b2.reshape(1, -1), W3, b3.reshape(1, -1))
    return out
